# baseline (speedup 1.0000x reference)
import functools

import jax
import jax.numpy as jnp
from jax import lax
from jax.experimental import pallas as pl
from jax.experimental.pallas import tpu as pltpu
from jax.experimental.pallas import tpu_sc as plsc

L = 16
F = 100000
FP = 100096
FPP = FP + 256
NF_T = FP // 16
NCH = NF_T // 16
NROW = NF_T // 128 + 1
NP = 208
PPT = NP // 16
HB = NP * 256
OB = PPT * 256
RN = 512
K_F, K_R = 64, 32
NQ = 256
QPT = NQ // 32
CONST_NO = 9999
NEG = -1e9
QCH = 256

_GDN = lax.GatherDimensionNumbers(
    offset_dims=(), collapsed_slice_dims=(0,), start_index_map=(0,))


def _iota():
    return lax.iota(jnp.int32, L)


def _bc(x):
    return jnp.full((L,), x, jnp.int32)


def _ds8(off, n):
    return pl.ds(pl.multiple_of(off, 8), n)


def _vperm(v, idx):
    return lax.gather(v, idx[:, None], _GDN, (1,),
                      mode=lax.GatherScatterMode.PROMISE_IN_BOUNDS)


def _last(v):
    return _vperm(v, _bc(L - 1))


def _shr1(v, carry):
    sh = _vperm(v, jnp.maximum(_iota() - 1, 0))
    return jnp.where(_iota() == 0, carry, sh)


def _better(aw, ai, bw, bi):
    return (aw > bw) | ((aw == bw) & (ai < bi))


def _insert(Tw, Ti, cw, ci):
    n = len(Tw)
    posb = _bc(0)
    for k in range(n):
        posb = posb + plsc.all_reduce_population_count(
            _better(Tw[k], Ti[k], cw, ci))
    nTw, nTi = [], []
    for k in range(n):
        lane = _iota() + L * k
        cwk = _last(Tw[k - 1]) if k else cw
        cik = _last(Ti[k - 1]) if k else ci
        shw = _shr1(Tw[k], cwk)
        shi = _shr1(Ti[k], cik)
        keep = lane < posb
        at = lane == posb
        nTw.append(jnp.where(keep, Tw[k], jnp.where(at, cw, shw)))
        nTi.append(jnp.where(keep, Ti[k], jnp.where(at, ci, shi)))
    return nTw, nTi


def _scan_group(Tw, Ti, wv, gv, match):
    n = len(Tw)
    worst_w = _last(Tw[-1])
    worst_i = _last(Ti[-1])
    acc0 = (match & _better(wv, gv, worst_w, worst_i)).astype(jnp.int32)

    def cond(st):
        return jnp.max(st[0]) > 0

    def body(st):
        acc = st[0]
        tw = list(st[1:1 + n])
        ti = list(st[1 + n:1 + 2 * n])
        posb = plsc.all_reduce_ffs(acc > 0)
        cw = _vperm(wv, posb)
        ci = _vperm(gv, posb)
        tw, ti = _insert(tw, ti, cw, ci)
        ww = _last(tw[-1])
        wi = _last(ti[-1])
        acc = jnp.where((_iota() != posb) & _better(wv, gv, ww, wi), acc, 0)
        return (acc, *tw, *ti)

    st = lax.while_loop(cond, body, (acc0, *Tw, *Ti))
    return list(st[1:1 + n]), list(st[1 + n:1 + 2 * n])


def _sc_body(fp_h, fa0_h, fa1_h, fw_h, rhp_h, rw_h, bod_h, goals_h,
             children_h, scores_h,
             fp_loc, fa0_loc, fa1_loc, fw_loc, gidx_loc, pos2d,
             hist, htall, cur_loc, bb, obuf, tot_loc, totall,
             idx2d, goals_loc,
             wbuf, a0buf, a1buf, gibuf, tidx, tfa0, tfa1, outbuf, scorebuf,
             rhp_loc, rw_loc, bod_loc,
             sh_hist, sh_tot, sh_cur, sp_w, sp_a0, sp_a1, sp_gi):
    c = lax.axis_index("c")
    s = lax.axis_index("s")
    fbase = s * NF_T

    pltpu.sync_copy(fp_h.at[_ds8(fbase, NF_T)], fp_loc.at[_ds8(0, NF_T)])
    pltpu.sync_copy(fa0_h.at[_ds8(fbase, NF_T)], fa0_loc.at[_ds8(0, NF_T)])
    pltpu.sync_copy(fa1_h.at[_ds8(fbase, NF_T)], fa1_loc.at[_ds8(0, NF_T)])
    pltpu.sync_copy(fw_h.at[_ds8(fbase, NF_T)], fw_loc.at[_ds8(0, NF_T)])
    pltpu.sync_copy(rhp_h, rhp_loc)
    pltpu.sync_copy(rw_h, rw_loc)
    pltpu.sync_copy(bod_h, bod_loc)
    pltpu.sync_copy(goals_h, goals_loc)

    def _zero(i, _):
        hist[_ds8(i * L, L)] = jnp.zeros((L,), jnp.int32)
        return 0
    lax.fori_loop(0, NP, _zero, 0)

    ones = jnp.ones((L,), jnp.int32)

    def _hbody(ch, _):
        pv = fp_loc[_ds8(ch * L, L)]
        hidx = pv * L + _iota()
        hv = plsc.load_gather(hist, [hidx])
        plsc.store_scatter(hist, [hidx], hv + ones)
        return 0
    lax.fori_loop(0, NCH, _hbody, 0)

    def _hidx(ch, _):
        idx2d[ch // 8, _ds8((ch % 8) * L, L)] = ch * 256 + s * L + _iota()
        return 0
    lax.fori_loop(0, NP, _hidx, 0)

    def _hpub(r, _):
        pltpu.sync_copy(hist.at[_ds8(r * 128, 128)], sh_hist.at[idx2d.at[r]])
        return 0
    lax.fori_loop(0, OB // 128, _hpub, 0)
    plsc.subcore_barrier()

    pltpu.sync_copy(sh_hist.at[_ds8(s * OB, OB)], htall)
    tot_loc[_ds8(0, L)] = jnp.zeros((L,), jnp.int32)
    for pp in range(PPT):
        run = _bc(0)
        for t in range(16):
            v = htall[_ds8(pp * 256 + t * L, L)]
            cums = plsc.cumsum(v)
            obuf[_ds8(pp * 256 + t * L, L)] = cums - v + run
            run = run + _last(cums)
        plsc.store_scatter(tot_loc, [_bc(pp)], run, mask=_iota() == 0)
    pltpu.sync_copy(tot_loc, sh_tot.at[_ds8(s * L, L)])
    plsc.subcore_barrier()

    pltpu.sync_copy(sh_tot, totall)
    carry = _bc(0)
    for r in range(16):
        v = totall[_ds8(r * L, L)]
        cums = plsc.cumsum(v)
        plsc.store_scatter(bb, [_bc(r * PPT) + _iota()], cums - v + carry,
                           mask=_iota() < PPT)
        carry = carry + _vperm(cums, _bc(PPT - 1))
    plsc.store_scatter(bb, [_bc(NP)], carry, mask=_iota() == 0)

    for pp in range(PPT):
        bbv = plsc.load_gather(bb, [s * PPT + _bc(pp)])
        for t in range(16):
            o = pp * 256 + t * L
            obuf[_ds8(o, L)] = obuf[_ds8(o, L)] + bbv

    def _cidx(ch, _):
        pp = ch // 16
        t = ch % 16
        idx2d[ch // 8, _ds8((ch % 8) * L, L)] = (
            t * OB + (s * PPT + pp) * L + _iota())
        return 0
    lax.fori_loop(0, OB // L, _cidx, 0)

    def _cpub(r, _):
        pltpu.sync_copy(obuf.at[_ds8(r * 128, 128)], sh_cur.at[idx2d.at[r]])
        return 0
    lax.fori_loop(0, OB // 128, _cpub, 0)
    plsc.subcore_barrier()

    pltpu.sync_copy(sh_cur.at[_ds8(s * OB, OB)], cur_loc)

    def _sbody(ch, _):
        pv = fp_loc[_ds8(ch * L, L)]
        hidx = pv * L + _iota()
        cur = plsc.load_gather(cur_loc, [hidx])
        pos2d[ch // 8, _ds8((ch % 8) * L, L)] = cur
        gidx_loc[_ds8(ch * L, L)] = fbase + ch * L + _iota()
        plsc.store_scatter(cur_loc, [hidx], cur + ones)
        return 0
    lax.fori_loop(0, NCH, _sbody, 0)
    pos2d[NCH // 8, _ds8((NCH % 8) * L, L)] = _bc(FP) + _iota()
    gidx_loc[_ds8(NF_T, L)] = _bc(0)

    def _scat(r, _):
        pltpu.sync_copy(fw_loc.at[_ds8(r * 128, 128)], sp_w.at[pos2d.at[r]])
        pltpu.sync_copy(fa0_loc.at[_ds8(r * 128, 128)], sp_a0.at[pos2d.at[r]])
        pltpu.sync_copy(fa1_loc.at[_ds8(r * 128, 128)], sp_a1.at[pos2d.at[r]])
        pltpu.sync_copy(gidx_loc.at[_ds8(r * 128, 128)], sp_gi.at[pos2d.at[r]])
        return 0
    lax.fori_loop(0, NROW, _scat, 0)
    plsc.subcore_barrier()

    qbase = (c * 16 + s) * QPT
    negw = jnp.full((L,), NEG, jnp.float32)
    maxi = _bc(0x7FFFFFFF)

    def _query(qi, _):
        qg = qbase + qi
        vlo = goals_loc[_ds8(qg * 32, L)]
        vhi = goals_loc[_ds8(qg * 32 + L, L)]
        qp = _vperm(vlo, _bc(0))
        qa0 = _vperm(vlo, _bc(1))
        qa1 = _vperm(vlo, _bc(2))
        var0 = qa0 > CONST_NO
        var1 = qa1 > CONST_NO
        rem = [_vperm(vlo, _bc(3 + j)) if 3 + j < L else
               _vperm(vhi, _bc(3 + j - L)) for j in range(21)]
        m0 = [var0 & (rem[j] == qa0) for j in range(21)]
        m1 = [var1 & (rem[j] == qa1) & ~m0[j] for j in range(21)]

        baseb = plsc.load_gather(bb, [qp])
        endb = plsc.load_gather(bb, [qp + 1])
        base_s = jnp.max(baseb)
        end_s = jnp.max(endb)
        astart = base_s & ~7
        nch = (end_s - astart + (QCH - 1)) // QCH

        Tw = [negw] * 4
        Ti = [maxi] * 4

        def _fchunk(cc, st):
            tw = list(st[:4])
            ti = list(st[4:])
            cstart = astart + cc * QCH
            pltpu.sync_copy(sp_w.at[_ds8(cstart, QCH)], wbuf)
            pltpu.sync_copy(sp_a0.at[_ds8(cstart, QCH)], a0buf)
            pltpu.sync_copy(sp_a1.at[_ds8(cstart, QCH)], a1buf)
            pltpu.sync_copy(sp_gi.at[_ds8(cstart, QCH)], gibuf)

            def _fgrp(i, st2):
                tw2 = list(st2[:4])
                ti2 = list(st2[4:])
                wv = wbuf[_ds8(i * L, L)]
                a0v = a0buf[_ds8(i * L, L)]
                a1v = a1buf[_ds8(i * L, L)]
                gv = gibuf[_ds8(i * L, L)]
                posv = cstart + i * L + _iota()
                inb = (posv >= baseb) & (posv < endb)
                m = inb & ((a0v == qa0) | var0) & ((a1v == qa1) | var1)
                tw2, ti2 = _scan_group(tw2, ti2, wv, gv, m)
                return (*tw2, *ti2)

            return lax.fori_loop(0, QCH // L, _fgrp, (*tw, *ti))

        st = lax.fori_loop(0, nch, _fchunk, (*Tw, *Ti))
        Tw = list(st[:4])
        Ti = list(st[4:])

        Rw = [negw] * 2
        Ri = [maxi] * 2

        def _rgrp(g, st2):
            rw2 = list(st2[:2])
            ri2 = list(st2[2:])
            pv = rhp_loc[_ds8(g * L, L)]
            wv = rw_loc[_ds8(g * L, L)]
            gv = g * L + _iota()
            rw2, ri2 = _scan_group(rw2, ri2, wv, gv, pv == qp)
            return (*rw2, *ri2)

        st2 = lax.fori_loop(0, RN // L, _rgrp, (*Rw, *Ri))
        Rw = list(st2[:2])
        Ri = list(st2[2:])

        for k in range(4):
            val = Tw[k] > NEG / 2
            tidx[_ds8(k * L, L)] = jnp.where(val, Ti[k], 0)
        pltpu.sync_copy(fa0_h.at[tidx], tfa0)
        pltpu.sync_copy(fa1_h.at[tidx], tfa1)

        for k in range(4):
            val = Tw[k] > NEG / 2
            b0 = tfa0[_ds8(k * L, L)]
            b1 = tfa1[_ds8(k * L, L)]
            row = (_iota() + k * L) * 24
            for j in range(21):
                col = jnp.where(m0[j], b0, jnp.where(m1[j], b1, rem[j]))
                plsc.store_scatter(outbuf, [row + j],
                                   jnp.where(val, col, 0))
            zero = jnp.zeros((L,), jnp.int32)
            for j in range(21, 24):
                plsc.store_scatter(outbuf, [row + j], zero)
            scorebuf[_ds8(k * L, L)] = Tw[k]
        for k in range(2):
            val = Rw[k] > NEG / 2
            ri = jnp.where(val, Ri[k], 0)
            row = (_iota() + K_F + k * L) * 24
            for j in range(9):
                col = plsc.load_gather(bod_loc, [ri * 9 + j])
                plsc.store_scatter(outbuf, [row + j],
                                   jnp.where(val, col, 0))
            for j in range(15):
                plsc.store_scatter(outbuf, [row + 9 + j],
                                   jnp.where(val, rem[j], 0))
            scorebuf[_ds8(K_F + k * L, L)] = Rw[k]

        pltpu.sync_copy(outbuf, children_h.at[_ds8(qg * 2304, 2304)])
        pltpu.sync_copy(scorebuf, scores_h.at[_ds8(qg * 96, 96)])
        return 0

    lax.fori_loop(0, QPT, _query, 0)


_SC_SCRATCH = [
    pltpu.VMEM((NROW * 128,), jnp.int32),
    pltpu.VMEM((NROW * 128,), jnp.int32),
    pltpu.VMEM((NROW * 128,), jnp.int32),
    pltpu.VMEM((NROW * 128,), jnp.float32),
    pltpu.VMEM((NROW * 128,), jnp.int32),
    pltpu.VMEM((NROW, 128), jnp.int32),
    pltpu.VMEM((NP * L,), jnp.int32),
    pltpu.VMEM((OB,), jnp.int32),
    pltpu.VMEM((OB,), jnp.int32),
    pltpu.VMEM((224,), jnp.int32),
    pltpu.VMEM((OB,), jnp.int32),
    pltpu.VMEM((L,), jnp.int32),
    pltpu.VMEM((256,), jnp.int32),
    pltpu.VMEM((OB // 128, 128), jnp.int32),
    pltpu.VMEM((NQ * 32,), jnp.int32),
    pltpu.VMEM((QCH,), jnp.float32),
    pltpu.VMEM((QCH,), jnp.int32),
    pltpu.VMEM((QCH,), jnp.int32),
    pltpu.VMEM((QCH,), jnp.int32),
    pltpu.VMEM((K_F,), jnp.int32),
    pltpu.VMEM((K_F,), jnp.int32),
    pltpu.VMEM((K_F,), jnp.int32),
    pltpu.VMEM((2304,), jnp.int32),
    pltpu.VMEM((96,), jnp.float32),
    pltpu.VMEM((RN,), jnp.int32),
    pltpu.VMEM((RN,), jnp.float32),
    pltpu.VMEM((RN * 9,), jnp.int32),
    pltpu.VMEM_SHARED((HB,), jnp.int32),
    pltpu.VMEM_SHARED((256,), jnp.int32),
    pltpu.VMEM_SHARED((HB,), jnp.int32),
    pltpu.VMEM_SHARED((FPP,), jnp.float32),
    pltpu.VMEM_SHARED((FPP,), jnp.int32),
    pltpu.VMEM_SHARED((FPP,), jnp.int32),
    pltpu.VMEM_SHARED((FPP,), jnp.int32),
]

_sc_call = functools.partial(
    pl.kernel,
    out_type=[jax.ShapeDtypeStruct((NQ * 2304,), jnp.int32),
              jax.ShapeDtypeStruct((NQ * 96,), jnp.float32)],
    mesh=plsc.VectorSubcoreMesh(core_axis_name="c", subcore_axis_name="s",
                                num_cores=2, num_subcores=16),
    scratch_types=_SC_SCRATCH,
    compiler_params=pltpu.CompilerParams(needs_layout_passes=False),
)(_sc_body)


def _lse_body(x_ref, o_ref):
    x = x_ref[...]
    m = jnp.max(x, axis=1, keepdims=True)
    o_ref[...] = m + jnp.log(jnp.sum(jnp.exp(x - m), axis=1, keepdims=True))


def kernel(facts_idx, rules_heads_idx, rules_bodies_idx, rule_lens,
           fact_weights, rule_weights, proof_goals):
    del rule_lens
    pad = FP - F
    fp = jnp.concatenate([facts_idx[:, 0], jnp.zeros((pad,), jnp.int32)])
    fa0 = jnp.concatenate([facts_idx[:, 1], jnp.zeros((pad,), jnp.int32)])
    fa1 = jnp.concatenate([facts_idx[:, 2], jnp.zeros((pad,), jnp.int32)])
    fw = jnp.concatenate([fact_weights, jnp.zeros((pad,), jnp.float32)])
    rhp = rules_heads_idx[:, 0]
    bod = rules_bodies_idx.reshape(RN * 9)
    goals = jnp.pad(proof_goals.reshape(NQ, 24), ((0, 0), (0, 8))).reshape(-1)

    children1d, cscores1d = _sc_call(fp, fa0, fa1, fw, rhp, rule_weights,
                                     bod, goals)
    cscores = cscores1d.reshape(NQ, 96)

    state = pl.pallas_call(
        _lse_body,
        out_shape=jax.ShapeDtypeStruct((NQ, 1), jnp.float32),
    )(cscores)

    children = children1d.reshape(8, 32, 96, 8, 3)
    child_scores = cscores.reshape(8, 32, 96)
    state_scores = state[:, 0].reshape(8, 32)
    return children, child_scores, state_scores


# speedup vs baseline: 1.0586x; 1.0586x over previous
import functools

import jax
import jax.numpy as jnp
from jax import lax
from jax.experimental import pallas as pl
from jax.experimental.pallas import tpu as pltpu
from jax.experimental.pallas import tpu_sc as plsc

L = 16
F = 100000
FP = 100096
FPP = FP + 256
NF_T = FP // 16
NCH = NF_T // 16
NROW = NF_T // 128 + 1
NP = 208
PPT = NP // 16
HB = NP * 256
OB = PPT * 256
RN = 512
K_F, K_R = 64, 32
NQ = 256
QPT = NQ // 32
CONST_NO = 9999
NEG = -1e9
QCH = 256

_GDN = lax.GatherDimensionNumbers(
    offset_dims=(), collapsed_slice_dims=(0,), start_index_map=(0,))


def _iota():
    return lax.iota(jnp.int32, L)


def _bc(x):
    return jnp.full((L,), x, jnp.int32)


def _ds8(off, n):
    return pl.ds(pl.multiple_of(off, 8), n)


def _vperm(v, idx):
    return lax.gather(v, idx[:, None], _GDN, (1,),
                      mode=lax.GatherScatterMode.PROMISE_IN_BOUNDS)


def _last(v):
    return _vperm(v, _bc(L - 1))


def _shr1(v, carry):
    sh = _vperm(v, jnp.maximum(_iota() - 1, 0))
    return jnp.where(_iota() == 0, carry, sh)


def _better(aw, ai, bw, bi):
    return (aw > bw) | ((aw == bw) & (ai < bi))


def _insert(Tw, Ti, cw, ci):
    n = len(Tw)
    cnt = _bc(0)
    for k in range(n):
        cnt = cnt + _better(Tw[k], Ti[k], cw, ci).astype(jnp.int32)
    posb = jnp.full((L,), jnp.sum(cnt), jnp.int32)
    nTw, nTi = [], []
    for k in range(n):
        lane = _iota() + L * k
        cwk = _last(Tw[k - 1]) if k else cw
        cik = _last(Ti[k - 1]) if k else ci
        shw = _shr1(Tw[k], cwk)
        shi = _shr1(Ti[k], cik)
        keep = lane < posb
        at = lane == posb
        nTw.append(jnp.where(keep, Tw[k], jnp.where(at, cw, shw)))
        nTi.append(jnp.where(keep, Ti[k], jnp.where(at, ci, shi)))
    return nTw, nTi


def _scan_group(Tw, Ti, wv, gv, match):
    n = len(Tw)
    worst_w = _last(Tw[-1])
    worst_i = _last(Ti[-1])
    acc0 = (match & _better(wv, gv, worst_w, worst_i)).astype(jnp.int32)

    def cond(st):
        return jnp.max(st[0]) > 0

    def body(st):
        acc = st[0]
        tw = list(st[1:1 + n])
        ti = list(st[1 + n:1 + 2 * n])
        posb = plsc.all_reduce_ffs(acc > 0)
        cw = _vperm(wv, posb)
        ci = _vperm(gv, posb)
        tw, ti = _insert(tw, ti, cw, ci)
        ww = _last(tw[-1])
        wi = _last(ti[-1])
        acc = jnp.where((_iota() != posb) & _better(wv, gv, ww, wi), acc, 0)
        return (acc, *tw, *ti)

    st = lax.while_loop(cond, body, (acc0, *Tw, *Ti))
    return list(st[1:1 + n]), list(st[1 + n:1 + 2 * n])


def _sc_body(fp_h, fa0_h, fa1_h, fw_h, rhp_h, rw_h, bod_h, goals_h,
             children_h, scores_h,
             fp_loc, fa0_loc, fa1_loc, fw_loc, gidx_loc, pos2d,
             hist, htall, cur_loc, bb, obuf, tot_loc, totall,
             idx2d, goals_loc,
             wbuf, a0buf, a1buf, gibuf, tidx, tfa0, tfa1, outbuf, scorebuf,
             rhp_loc, rw_loc, bod_loc, qcnt, dsem,
             sh_hist, sh_tot, sh_cur, sp_w, sp_a0, sp_a1, sp_gi):
    c = lax.axis_index("c")
    s = lax.axis_index("s")
    fbase = s * NF_T

    pltpu.sync_copy(fp_h.at[_ds8(fbase, NF_T)], fp_loc.at[_ds8(0, NF_T)])
    pltpu.sync_copy(fa0_h.at[_ds8(fbase, NF_T)], fa0_loc.at[_ds8(0, NF_T)])
    pltpu.sync_copy(fa1_h.at[_ds8(fbase, NF_T)], fa1_loc.at[_ds8(0, NF_T)])
    pltpu.sync_copy(fw_h.at[_ds8(fbase, NF_T)], fw_loc.at[_ds8(0, NF_T)])
    pltpu.sync_copy(rhp_h, rhp_loc)
    pltpu.sync_copy(rw_h, rw_loc)
    pltpu.sync_copy(bod_h, bod_loc)
    pltpu.sync_copy(goals_h, goals_loc)

    def _zero(i, _):
        hist[_ds8(i * L, L)] = jnp.zeros((L,), jnp.int32)
        return 0
    lax.fori_loop(0, NP, _zero, 0)

    ones = jnp.ones((L,), jnp.int32)

    def _hbody(ch, _):
        pv = fp_loc[_ds8(ch * L, L)]
        hidx = pv * L + _iota()
        hv = plsc.load_gather(hist, [hidx])
        plsc.store_scatter(hist, [hidx], hv + ones)
        return 0
    lax.fori_loop(0, NCH, _hbody, 0)

    def _hidx(ch, _):
        idx2d[ch // 8, _ds8((ch % 8) * L, L)] = ch * 256 + s * L + _iota()
        return 0
    lax.fori_loop(0, NP, _hidx, 0)

    def _hfire(r, _):
        pltpu.async_copy(hist.at[_ds8(r * 128, 128)],
                         sh_hist.at[idx2d.at[r]], dsem)
        return 0
    lax.fori_loop(0, OB // 128, _hfire, 0)

    def _hdrain(r, _):
        pltpu.make_async_copy(hist.at[_ds8(r * 128, 128)],
                              sh_hist.at[idx2d.at[r]], dsem).wait()
        return 0
    lax.fori_loop(0, OB // 128, _hdrain, 0)
    plsc.subcore_barrier()

    pltpu.sync_copy(sh_hist.at[_ds8(s * OB, OB)], htall)
    tot_loc[_ds8(0, L)] = jnp.zeros((L,), jnp.int32)
    for pp in range(PPT):
        run = _bc(0)
        for t in range(16):
            v = htall[_ds8(pp * 256 + t * L, L)]
            cums = plsc.cumsum(v)
            obuf[_ds8(pp * 256 + t * L, L)] = cums - v + run
            run = run + _last(cums)
        plsc.store_scatter(tot_loc, [_bc(pp)], run, mask=_iota() == 0)
    pltpu.sync_copy(tot_loc, sh_tot.at[_ds8(s * L, L)])
    plsc.subcore_barrier()

    pltpu.sync_copy(sh_tot, totall)
    carry = _bc(0)
    for r in range(16):
        v = totall[_ds8(r * L, L)]
        cums = plsc.cumsum(v)
        plsc.store_scatter(bb, [_bc(r * PPT) + _iota()], cums - v + carry,
                           mask=_iota() < PPT)
        carry = carry + _vperm(cums, _bc(PPT - 1))
    plsc.store_scatter(bb, [_bc(NP)], carry, mask=_iota() == 0)

    for pp in range(PPT):
        bbv = plsc.load_gather(bb, [s * PPT + _bc(pp)])
        for t in range(16):
            o = pp * 256 + t * L
            obuf[_ds8(o, L)] = obuf[_ds8(o, L)] + bbv

    def _cidx(ch, _):
        pp = ch // 16
        t = ch % 16
        idx2d[ch // 8, _ds8((ch % 8) * L, L)] = (
            t * OB + (s * PPT + pp) * L + _iota())
        return 0
    lax.fori_loop(0, OB // L, _cidx, 0)

    def _cpub(r, _):
        pltpu.sync_copy(obuf.at[_ds8(r * 128, 128)], sh_cur.at[idx2d.at[r]])
        return 0
    lax.fori_loop(0, OB // 128, _cpub, 0)
    plsc.subcore_barrier()

    pltpu.sync_copy(sh_cur.at[_ds8(s * OB, OB)], cur_loc)

    def _sbody(ch, _):
        pv = fp_loc[_ds8(ch * L, L)]
        hidx = pv * L + _iota()
        cur = plsc.load_gather(cur_loc, [hidx])
        pos2d[ch // 8, _ds8((ch % 8) * L, L)] = cur
        gidx_loc[_ds8(ch * L, L)] = fbase + ch * L + _iota()
        plsc.store_scatter(cur_loc, [hidx], cur + ones)
        return 0
    lax.fori_loop(0, NCH, _sbody, 0)
    pos2d[NCH // 8, _ds8((NCH % 8) * L, L)] = _bc(FP) + _iota()
    gidx_loc[_ds8(NF_T, L)] = _bc(0)

    def _sfire(r, _):
        pltpu.async_copy(fw_loc.at[_ds8(r * 128, 128)], sp_w.at[pos2d.at[r]], dsem)
        pltpu.async_copy(fa0_loc.at[_ds8(r * 128, 128)], sp_a0.at[pos2d.at[r]], dsem)
        pltpu.async_copy(fa1_loc.at[_ds8(r * 128, 128)], sp_a1.at[pos2d.at[r]], dsem)
        pltpu.async_copy(gidx_loc.at[_ds8(r * 128, 128)], sp_gi.at[pos2d.at[r]], dsem)
        return 0
    lax.fori_loop(0, NROW, _sfire, 0)

    def _sdrain(r, _):
        pltpu.make_async_copy(fw_loc.at[_ds8(r * 128, 128)],
                              sp_w.at[pos2d.at[r]], dsem).wait()
        pltpu.make_async_copy(fa0_loc.at[_ds8(r * 128, 128)],
                              sp_a0.at[pos2d.at[r]], dsem).wait()
        pltpu.make_async_copy(fa1_loc.at[_ds8(r * 128, 128)],
                              sp_a1.at[pos2d.at[r]], dsem).wait()
        pltpu.make_async_copy(gidx_loc.at[_ds8(r * 128, 128)],
                              sp_gi.at[pos2d.at[r]], dsem).wait()
        return 0
    lax.fori_loop(0, NROW, _sdrain, 0)
    @pl.when(s == 0)
    def _():
        qcnt[0] = 0
    plsc.subcore_barrier()

    negw = jnp.full((L,), NEG, jnp.float32)
    maxi = _bc(0x7FFFFFFF)

    def _query(qg):
        vlo = goals_loc[_ds8(qg * 32, L)]
        vhi = goals_loc[_ds8(qg * 32 + L, L)]
        qp = _vperm(vlo, _bc(0))
        qa0 = _vperm(vlo, _bc(1))
        qa1 = _vperm(vlo, _bc(2))
        var0 = qa0 > CONST_NO
        var1 = qa1 > CONST_NO
        rem = [_vperm(vlo, _bc(3 + j)) if 3 + j < L else
               _vperm(vhi, _bc(3 + j - L)) for j in range(21)]
        m0 = [var0 & (rem[j] == qa0) for j in range(21)]
        m1 = [var1 & (rem[j] == qa1) & ~m0[j] for j in range(21)]

        baseb = plsc.load_gather(bb, [qp])
        endb = plsc.load_gather(bb, [qp + 1])
        base_s = jnp.max(baseb)
        end_s = jnp.max(endb)
        astart = base_s & ~7
        nch = (end_s - astart + (QCH - 1)) // QCH

        Tw = [negw] * 4
        Ti = [maxi] * 4

        def _fchunk(cc, st):
            tw = list(st[:4])
            ti = list(st[4:])
            cstart = astart + cc * QCH
            pltpu.async_copy(sp_w.at[_ds8(cstart, QCH)], wbuf, dsem)
            pltpu.async_copy(sp_a0.at[_ds8(cstart, QCH)], a0buf, dsem)
            pltpu.async_copy(sp_a1.at[_ds8(cstart, QCH)], a1buf, dsem)
            pltpu.async_copy(sp_gi.at[_ds8(cstart, QCH)], gibuf, dsem)
            pltpu.make_async_copy(sp_w.at[_ds8(cstart, QCH)], wbuf, dsem).wait()
            pltpu.make_async_copy(sp_a0.at[_ds8(cstart, QCH)], a0buf, dsem).wait()
            pltpu.make_async_copy(sp_a1.at[_ds8(cstart, QCH)], a1buf, dsem).wait()
            pltpu.make_async_copy(sp_gi.at[_ds8(cstart, QCH)], gibuf, dsem).wait()
            ngrp = jnp.minimum(QCH // L,
                               jnp.maximum(0, (end_s - cstart + L - 1) // L))

            def _fgrp(i, st2):
                tw2 = list(st2[:4])
                ti2 = list(st2[4:])
                wv = wbuf[_ds8(i * L, L)]
                a0v = a0buf[_ds8(i * L, L)]
                a1v = a1buf[_ds8(i * L, L)]
                gv = gibuf[_ds8(i * L, L)]
                posv = cstart + i * L + _iota()
                inb = (posv >= baseb) & (posv < endb)
                m = inb & ((a0v == qa0) | var0) & ((a1v == qa1) | var1)
                tw2, ti2 = _scan_group(tw2, ti2, wv, gv, m)
                return (*tw2, *ti2)

            return lax.fori_loop(0, ngrp, _fgrp, (*tw, *ti))

        st = lax.fori_loop(0, nch, _fchunk, (*Tw, *Ti))
        Tw = list(st[:4])
        Ti = list(st[4:])

        Rw = [negw] * 2
        Ri = [maxi] * 2

        def _rgrp(g, st2):
            rw2 = list(st2[:2])
            ri2 = list(st2[2:])
            pv = rhp_loc[_ds8(g * L, L)]
            wv = rw_loc[_ds8(g * L, L)]
            gv = g * L + _iota()
            rw2, ri2 = _scan_group(rw2, ri2, wv, gv, pv == qp)
            return (*rw2, *ri2)

        st2 = lax.fori_loop(0, RN // L, _rgrp, (*Rw, *Ri))
        Rw = list(st2[:2])
        Ri = list(st2[2:])

        for k in range(4):
            val = Tw[k] > NEG / 2
            tidx[_ds8(k * L, L)] = jnp.where(val, Ti[k], 0)
        pltpu.sync_copy(fa0_h.at[tidx], tfa0)
        pltpu.sync_copy(fa1_h.at[tidx], tfa1)

        for k in range(4):
            val = Tw[k] > NEG / 2
            b0 = tfa0[_ds8(k * L, L)]
            b1 = tfa1[_ds8(k * L, L)]
            row = (_iota() + k * L) * 24
            for j in range(21):
                col = jnp.where(m0[j], b0, jnp.where(m1[j], b1, rem[j]))
                plsc.store_scatter(outbuf, [row + j],
                                   jnp.where(val, col, 0))
            zero = jnp.zeros((L,), jnp.int32)
            for j in range(21, 24):
                plsc.store_scatter(outbuf, [row + j], zero)
            scorebuf[_ds8(k * L, L)] = Tw[k]
        for k in range(2):
            val = Rw[k] > NEG / 2
            ri = jnp.where(val, Ri[k], 0)
            row = (_iota() + K_F + k * L) * 24
            for j in range(9):
                col = plsc.load_gather(bod_loc, [ri * 9 + j])
                plsc.store_scatter(outbuf, [row + j],
                                   jnp.where(val, col, 0))
            for j in range(15):
                plsc.store_scatter(outbuf, [row + 9 + j],
                                   jnp.where(val, rem[j], 0))
            scorebuf[_ds8(K_F + k * L, L)] = Rw[k]

        pltpu.sync_copy(outbuf, children_h.at[_ds8(qg * 2304, 2304)])
        pltpu.sync_copy(scorebuf, scores_h.at[_ds8(qg * 96, 96)])

    half = NQ // 2

    def _qcond(q):
        return q < half

    def _qbody(q):
        _query(c * half + q)
        return plsc.fetch_and_add(qcnt.at[0], 1, subcore_id=0)

    lax.while_loop(_qcond, _qbody,
                   plsc.fetch_and_add(qcnt.at[0], 1, subcore_id=0))


_SC_SCRATCH = [
    pltpu.VMEM((NROW * 128,), jnp.int32),
    pltpu.VMEM((NROW * 128,), jnp.int32),
    pltpu.VMEM((NROW * 128,), jnp.int32),
    pltpu.VMEM((NROW * 128,), jnp.float32),
    pltpu.VMEM((NROW * 128,), jnp.int32),
    pltpu.VMEM((NROW, 128), jnp.int32),
    pltpu.VMEM((NP * L,), jnp.int32),
    pltpu.VMEM((OB,), jnp.int32),
    pltpu.VMEM((OB,), jnp.int32),
    pltpu.VMEM((224,), jnp.int32),
    pltpu.VMEM((OB,), jnp.int32),
    pltpu.VMEM((L,), jnp.int32),
    pltpu.VMEM((256,), jnp.int32),
    pltpu.VMEM((OB // 128, 128), jnp.int32),
    pltpu.VMEM((NQ * 32,), jnp.int32),
    pltpu.VMEM((QCH,), jnp.float32),
    pltpu.VMEM((QCH,), jnp.int32),
    pltpu.VMEM((QCH,), jnp.int32),
    pltpu.VMEM((QCH,), jnp.int32),
    pltpu.VMEM((K_F,), jnp.int32),
    pltpu.VMEM((K_F,), jnp.int32),
    pltpu.VMEM((K_F,), jnp.int32),
    pltpu.VMEM((2304,), jnp.int32),
    pltpu.VMEM((96,), jnp.float32),
    pltpu.VMEM((RN,), jnp.int32),
    pltpu.VMEM((RN,), jnp.float32),
    pltpu.VMEM((RN * 9,), jnp.int32),
    pltpu.SMEM((1,), jnp.int32),
    pltpu.SemaphoreType.DMA,
    pltpu.VMEM_SHARED((HB,), jnp.int32),
    pltpu.VMEM_SHARED((256,), jnp.int32),
    pltpu.VMEM_SHARED((HB,), jnp.int32),
    pltpu.VMEM_SHARED((FPP,), jnp.float32),
    pltpu.VMEM_SHARED((FPP,), jnp.int32),
    pltpu.VMEM_SHARED((FPP,), jnp.int32),
    pltpu.VMEM_SHARED((FPP,), jnp.int32),
]

_sc_call = functools.partial(
    pl.kernel,
    out_type=[jax.ShapeDtypeStruct((NQ * 2304,), jnp.int32),
              jax.ShapeDtypeStruct((NQ * 96,), jnp.float32)],
    mesh=plsc.VectorSubcoreMesh(core_axis_name="c", subcore_axis_name="s",
                                num_cores=2, num_subcores=16),
    scratch_types=_SC_SCRATCH,
    compiler_params=pltpu.CompilerParams(needs_layout_passes=False),
)(_sc_body)


def _lse_body(x_ref, o_ref):
    x = x_ref[...]
    m = jnp.max(x, axis=1, keepdims=True)
    o_ref[...] = m + jnp.log(jnp.sum(jnp.exp(x - m), axis=1, keepdims=True))


def kernel(facts_idx, rules_heads_idx, rules_bodies_idx, rule_lens,
           fact_weights, rule_weights, proof_goals):
    del rule_lens
    pad = FP - F
    fp = jnp.concatenate([facts_idx[:, 0], jnp.zeros((pad,), jnp.int32)])
    fa0 = jnp.concatenate([facts_idx[:, 1], jnp.zeros((pad,), jnp.int32)])
    fa1 = jnp.concatenate([facts_idx[:, 2], jnp.zeros((pad,), jnp.int32)])
    fw = jnp.concatenate([fact_weights, jnp.zeros((pad,), jnp.float32)])
    rhp = rules_heads_idx[:, 0]
    bod = rules_bodies_idx.reshape(RN * 9)
    goals = jnp.pad(proof_goals.reshape(NQ, 24), ((0, 0), (0, 8))).reshape(-1)

    children1d, cscores1d = _sc_call(fp, fa0, fa1, fw, rhp, rule_weights,
                                     bod, goals)
    cscores = cscores1d.reshape(NQ, 96)

    state = pl.pallas_call(
        _lse_body,
        out_shape=jax.ShapeDtypeStruct((NQ, 1), jnp.float32),
    )(cscores)

    children = children1d.reshape(8, 32, 96, 8, 3)
    child_scores = cscores.reshape(8, 32, 96)
    state_scores = state[:, 0].reshape(8, 32)
    return children, child_scores, state_scores
